# baseline (speedup 1.0000x reference)
import jax
import jax.numpy as jnp
from jax.experimental import pallas as pl
from jax.experimental.pallas import tpu as pltpu

N = 10000
D = 128
H = 128
EPS = 1e-5

BR = 200
NC = N // BR
R = 4


def _fused_kernel(adj_hbm, h_ref, W1_ref, b1_ref, W2_ref, b2_ref,
                  gamma_ref, beta_ref, Wp_ref, bp_ref,
                  out_ref, x_scr, buf_scr, sems):

    def chunk_copy(i, s):
        return pltpu.make_async_copy(
            adj_hbm.at[pl.ds(i * BR, BR), :], buf_scr.at[s], sems.at[s])

    for s in range(R):
        chunk_copy(s, s).start()

    def body(i, carry):
        s_sum, s_ssq = carry
        sl = jax.lax.rem(i, R)
        chunk_copy(i, sl).wait()

        pooled = jnp.dot(buf_scr[sl].astype(jnp.bfloat16), h_ref[...],
                         preferred_element_type=jnp.float32)

        @pl.when(i + R < NC)
        def _():
            chunk_copy(i + R, sl).start()

        x = jnp.maximum(
            jnp.dot(pooled, W1_ref[...], preferred_element_type=jnp.float32)
            + b1_ref[0, :], 0.0)
        x = (jnp.dot(x, W2_ref[...], preferred_element_type=jnp.float32)
             + b2_ref[0, :])
        x_scr[pl.ds(i * BR, BR), :] = x
        return (s_sum + jnp.sum(x, axis=0, keepdims=True),
                s_ssq + jnp.sum(x * x, axis=0, keepdims=True))

    zero = jnp.zeros((1, H), jnp.float32)
    s_sum, s_ssq = jax.lax.fori_loop(0, NC, body, (zero, zero))

    m = s_sum[0, :] * (1.0 / N)
    v = s_ssq[0, :] * (1.0 / N) - m * m
    inv = jax.lax.rsqrt(v + EPS)
    scale = gamma_ref[0, :] * inv
    shift = beta_ref[0, :] - m * scale
    y = jnp.maximum(x_scr[...] * scale + shift, 0.0)
    out_ref[...] = (jnp.dot(y, Wp_ref[...], preferred_element_type=jnp.float32)
                    + bp_ref[0, 0])


@jax.jit
def kernel(seq1, adj, W1, b1, W2, b2, gamma, beta, Wp, bp):
    out = pl.pallas_call(
        _fused_kernel,
        in_specs=[
            pl.BlockSpec(memory_space=pltpu.MemorySpace.HBM),
            pl.BlockSpec(memory_space=pltpu.MemorySpace.VMEM),
            pl.BlockSpec(memory_space=pltpu.MemorySpace.VMEM),
            pl.BlockSpec(memory_space=pltpu.MemorySpace.VMEM),
            pl.BlockSpec(memory_space=pltpu.MemorySpace.VMEM),
            pl.BlockSpec(memory_space=pltpu.MemorySpace.VMEM),
            pl.BlockSpec(memory_space=pltpu.MemorySpace.VMEM),
            pl.BlockSpec(memory_space=pltpu.MemorySpace.VMEM),
            pl.BlockSpec(memory_space=pltpu.MemorySpace.VMEM),
            pl.BlockSpec(memory_space=pltpu.MemorySpace.VMEM),
        ],
        out_specs=pl.BlockSpec(memory_space=pltpu.MemorySpace.VMEM),
        out_shape=jax.ShapeDtypeStruct((N, 1), jnp.float32),
        scratch_shapes=[
            pltpu.VMEM((N, H), jnp.float32),
            pltpu.VMEM((R, BR, N), jnp.float32),
            pltpu.SemaphoreType.DMA((R,)),
        ],
    )(adj, seq1.astype(jnp.bfloat16), W1, b1.reshape(1, H),
      W2, b2.reshape(1, H), gamma.reshape(1, H), beta.reshape(1, H),
      Wp, bp.reshape(1, 1))
    return out


# speedup vs baseline: 1.0352x; 1.0352x over previous
import jax
import jax.numpy as jnp
from jax.experimental import pallas as pl
from jax.experimental.pallas import tpu as pltpu

N = 10000
D = 128
H = 128
EPS = 1e-5

BR = 400
HB = BR // 2
NC = N // BR
R = 3


def _fused_kernel(adj_hbm, h_ref, W1_ref, b1_ref, W2_ref, b2_ref,
                  gamma_ref, beta_ref, Wp_ref, bp_ref,
                  out_ref, x_scr, buf_scr, sems):

    def half_copy(i, s, hh):
        return pltpu.make_async_copy(
            adj_hbm.at[pl.ds(i * BR + hh * HB, HB), :],
            buf_scr.at[s, pl.ds(hh * HB, HB), :],
            sems.at[s, hh])

    def start_chunk(i, s):
        half_copy(i, s, 0).start()
        half_copy(i, s, 1).start()

    def wait_chunk(i, s):
        half_copy(i, s, 0).wait()
        half_copy(i, s, 1).wait()

    for s in range(R):
        start_chunk(s, s)

    def body(i, carry):
        s_sum, s_ssq = carry
        sl = jax.lax.rem(i, R)
        wait_chunk(i, sl)

        pooled = jnp.dot(buf_scr[sl].astype(jnp.bfloat16), h_ref[...],
                         preferred_element_type=jnp.float32)

        @pl.when(i + R < NC)
        def _():
            start_chunk(i + R, sl)

        x = jnp.maximum(
            jnp.dot(pooled, W1_ref[...], preferred_element_type=jnp.float32)
            + b1_ref[0, :], 0.0)
        x = (jnp.dot(x, W2_ref[...], preferred_element_type=jnp.float32)
             + b2_ref[0, :])
        x_scr[pl.ds(i * BR, BR), :] = x.astype(jnp.bfloat16)
        return (s_sum + jnp.sum(x, axis=0, keepdims=True),
                s_ssq + jnp.sum(x * x, axis=0, keepdims=True))

    zero = jnp.zeros((1, H), jnp.float32)
    s_sum, s_ssq = jax.lax.fori_loop(0, NC, body, (zero, zero))

    m = s_sum[0, :] * (1.0 / N)
    v = s_ssq[0, :] * (1.0 / N) - m * m
    inv = jax.lax.rsqrt(v + EPS)
    scale = gamma_ref[0, :] * inv
    shift = beta_ref[0, :] - m * scale
    y = jnp.maximum(x_scr[...].astype(jnp.float32) * scale + shift, 0.0)
    out_ref[...] = jax.lax.dot_general(
        Wp_ref[...], y, (((1,), (1,)), ((), ())),
        preferred_element_type=jnp.float32) + bp_ref[0, 0]


@jax.jit
def kernel(seq1, adj, W1, b1, W2, b2, gamma, beta, Wp, bp):
    out = pl.pallas_call(
        _fused_kernel,
        in_specs=[
            pl.BlockSpec(memory_space=pltpu.MemorySpace.HBM),
            pl.BlockSpec(memory_space=pltpu.MemorySpace.VMEM),
            pl.BlockSpec(memory_space=pltpu.MemorySpace.VMEM),
            pl.BlockSpec(memory_space=pltpu.MemorySpace.VMEM),
            pl.BlockSpec(memory_space=pltpu.MemorySpace.VMEM),
            pl.BlockSpec(memory_space=pltpu.MemorySpace.VMEM),
            pl.BlockSpec(memory_space=pltpu.MemorySpace.VMEM),
            pl.BlockSpec(memory_space=pltpu.MemorySpace.VMEM),
            pl.BlockSpec(memory_space=pltpu.MemorySpace.VMEM),
            pl.BlockSpec(memory_space=pltpu.MemorySpace.VMEM),
        ],
        out_specs=pl.BlockSpec(memory_space=pltpu.MemorySpace.VMEM),
        out_shape=jax.ShapeDtypeStruct((1, N), jnp.float32),
        scratch_shapes=[
            pltpu.VMEM((N, H), jnp.bfloat16),
            pltpu.VMEM((R, BR, N), jnp.float32),
            pltpu.SemaphoreType.DMA((R, 2)),
        ],
    )(adj, seq1.astype(jnp.bfloat16), W1, b1.reshape(1, H),
      W2, b2.reshape(1, H), gamma.reshape(1, H), beta.reshape(1, H),
      Wp.reshape(1, H), bp.reshape(1, 1))
    return out.reshape(N, 1)


# speedup vs baseline: 1.0566x; 1.0207x over previous
import jax
import jax.numpy as jnp
from jax.experimental import pallas as pl
from jax.experimental.pallas import tpu as pltpu

N = 10000
D = 128
H = 128
EPS = 1e-5

BR = 400
HB = BR // 2
NC = N // BR
R = 3


def _fused_kernel(adj_hbm, h_ref, W1_ref, b1_ref, W2_ref, b2_ref,
                  gamma_ref, beta_ref, Wp_ref, bp_ref,
                  out_ref, x_scr, buf_scr, sems):

    def half_copy(i, s, hh):
        return pltpu.make_async_copy(
            adj_hbm.at[pl.ds(i * BR + hh * HB, HB), :],
            buf_scr.at[s, pl.ds(hh * HB, HB), :],
            sems.at[s, hh])

    def start_chunk(i, s):
        half_copy(i, s, 0).start()
        half_copy(i, s, 1).start()

    def wait_chunk(i, s):
        half_copy(i, s, 0).wait()
        half_copy(i, s, 1).wait()

    for s in range(R):
        start_chunk(s, s)

    def body(i, carry):
        s_sum, s_ssq = carry
        sl = jax.lax.rem(i, R)
        wait_chunk(i, sl)

        pooled = jnp.dot(buf_scr[sl], h_ref[...],
                         preferred_element_type=jnp.float32)

        @pl.when(i + R < NC)
        def _():
            start_chunk(i + R, sl)

        x = jnp.maximum(
            jnp.dot(pooled, W1_ref[...], preferred_element_type=jnp.float32)
            + b1_ref[0, :], 0.0)
        x = (jnp.dot(x, W2_ref[...], preferred_element_type=jnp.float32)
             + b2_ref[0, :])
        x_scr[pl.ds(i * BR, BR), :] = x
        return (s_sum + jnp.sum(x, axis=0, keepdims=True),
                s_ssq + jnp.sum(x * x, axis=0, keepdims=True))

    zero = jnp.zeros((1, H), jnp.float32)
    s_sum, s_ssq = jax.lax.fori_loop(0, NC, body, (zero, zero))

    m = s_sum[0, :] * (1.0 / N)
    v = s_ssq[0, :] * (1.0 / N) - m * m
    inv = jax.lax.rsqrt(v + EPS)
    scale = gamma_ref[0, :] * inv
    shift = beta_ref[0, :] - m * scale
    y = jnp.maximum(x_scr[...] * scale + shift, 0.0)
    out_ref[...] = jax.lax.dot_general(
        Wp_ref[...], y, (((1,), (1,)), ((), ())),
        preferred_element_type=jnp.float32) + bp_ref[0, 0]


@jax.jit
def kernel(seq1, adj, W1, b1, W2, b2, gamma, beta, Wp, bp):
    out = pl.pallas_call(
        _fused_kernel,
        in_specs=[
            pl.BlockSpec(memory_space=pltpu.MemorySpace.HBM),
            pl.BlockSpec(memory_space=pltpu.MemorySpace.VMEM),
            pl.BlockSpec(memory_space=pltpu.MemorySpace.VMEM),
            pl.BlockSpec(memory_space=pltpu.MemorySpace.VMEM),
            pl.BlockSpec(memory_space=pltpu.MemorySpace.VMEM),
            pl.BlockSpec(memory_space=pltpu.MemorySpace.VMEM),
            pl.BlockSpec(memory_space=pltpu.MemorySpace.VMEM),
            pl.BlockSpec(memory_space=pltpu.MemorySpace.VMEM),
            pl.BlockSpec(memory_space=pltpu.MemorySpace.VMEM),
            pl.BlockSpec(memory_space=pltpu.MemorySpace.VMEM),
        ],
        out_specs=pl.BlockSpec(memory_space=pltpu.MemorySpace.VMEM),
        out_shape=jax.ShapeDtypeStruct((1, N), jnp.float32),
        scratch_shapes=[
            pltpu.VMEM((N, H), jnp.float32),
            pltpu.VMEM((R, BR, N), jnp.float32),
            pltpu.SemaphoreType.DMA((R, 2)),
        ],
    )(adj, seq1, W1, b1.reshape(1, H),
      W2, b2.reshape(1, H), gamma.reshape(1, H), beta.reshape(1, H),
      Wp.reshape(1, H), bp.reshape(1, 1))
    return out.reshape(N, 1)


# speedup vs baseline: 1.0599x; 1.0031x over previous
import jax
import jax.numpy as jnp
from jax.experimental import pallas as pl
from jax.experimental.pallas import tpu as pltpu

N = 10000
D = 128
H = 128
EPS = 1e-5

BR = 400
NQ = 5
HB = BR // NQ
NC = N // BR
R = 3


def _fused_kernel(adj_hbm, h_ref, W1_ref, b1_ref, W2_ref, b2_ref,
                  gamma_ref, beta_ref, Wp_ref, bp_ref,
                  out_ref, x_scr, buf_scr, sems):

    def half_copy(i, s, hh):
        return pltpu.make_async_copy(
            adj_hbm.at[pl.ds(i * BR + hh * HB, HB), :],
            buf_scr.at[s, pl.ds(hh * HB, HB), :],
            sems.at[s, hh])

    def start_chunk(i, s):
        for hh in range(NQ):
            half_copy(i, s, hh).start()

    def wait_chunk(i, s):
        for hh in range(NQ):
            half_copy(i, s, hh).wait()

    for s in range(R):
        start_chunk(s, s)

    def body(i, carry):
        s_sum, s_ssq = carry
        sl = jax.lax.rem(i, R)
        wait_chunk(i, sl)

        pooled = jnp.dot(buf_scr[sl], h_ref[...],
                         preferred_element_type=jnp.float32)

        @pl.when(i + R < NC)
        def _():
            start_chunk(i + R, sl)

        x = jnp.maximum(
            jnp.dot(pooled, W1_ref[...], preferred_element_type=jnp.float32)
            + b1_ref[0, :], 0.0)
        x = (jnp.dot(x, W2_ref[...], preferred_element_type=jnp.float32)
             + b2_ref[0, :])
        x_scr[pl.ds(i * BR, BR), :] = x
        return (s_sum + jnp.sum(x, axis=0, keepdims=True),
                s_ssq + jnp.sum(x * x, axis=0, keepdims=True))

    zero = jnp.zeros((1, H), jnp.float32)
    s_sum, s_ssq = jax.lax.fori_loop(0, NC, body, (zero, zero))

    m = s_sum[0, :] * (1.0 / N)
    v = s_ssq[0, :] * (1.0 / N) - m * m
    inv = jax.lax.rsqrt(v + EPS)
    scale = gamma_ref[0, :] * inv
    shift = beta_ref[0, :] - m * scale
    y = jnp.maximum(x_scr[...] * scale + shift, 0.0)
    out_ref[...] = jax.lax.dot_general(
        Wp_ref[...], y, (((1,), (1,)), ((), ())),
        preferred_element_type=jnp.float32) + bp_ref[0, 0]


@jax.jit
def kernel(seq1, adj, W1, b1, W2, b2, gamma, beta, Wp, bp):
    out = pl.pallas_call(
        _fused_kernel,
        in_specs=[
            pl.BlockSpec(memory_space=pltpu.MemorySpace.HBM),
            pl.BlockSpec(memory_space=pltpu.MemorySpace.VMEM),
            pl.BlockSpec(memory_space=pltpu.MemorySpace.VMEM),
            pl.BlockSpec(memory_space=pltpu.MemorySpace.VMEM),
            pl.BlockSpec(memory_space=pltpu.MemorySpace.VMEM),
            pl.BlockSpec(memory_space=pltpu.MemorySpace.VMEM),
            pl.BlockSpec(memory_space=pltpu.MemorySpace.VMEM),
            pl.BlockSpec(memory_space=pltpu.MemorySpace.VMEM),
            pl.BlockSpec(memory_space=pltpu.MemorySpace.VMEM),
            pl.BlockSpec(memory_space=pltpu.MemorySpace.VMEM),
        ],
        out_specs=pl.BlockSpec(memory_space=pltpu.MemorySpace.VMEM),
        out_shape=jax.ShapeDtypeStruct((1, N), jnp.float32),
        scratch_shapes=[
            pltpu.VMEM((N, H), jnp.float32),
            pltpu.VMEM((R, BR, N), jnp.float32),
            pltpu.SemaphoreType.DMA((R, NQ)),
        ],
    )(adj, seq1, W1, b1.reshape(1, H),
      W2, b2.reshape(1, H), gamma.reshape(1, H), beta.reshape(1, H),
      Wp.reshape(1, H), bp.reshape(1, 1))
    return out.reshape(N, 1)
